# Initial kernel scaffold; baseline (speedup 1.0000x reference)
import jax
import jax.numpy as jnp
from jax.experimental import pallas as pl


def kernel(hidden_states, gate_w, w1, w2):
    raise NotImplementedError("write your pallas kernel here")


# speedup vs baseline: 1.3629x; 1.3629x over previous
import functools

import jax
import jax.numpy as jnp
from jax.experimental import pallas as pl
from jax.experimental.pallas import tpu as pltpu

T, H, F, E, TOP_K = 2048, 1024, 1024, 8, 2


def _router_body(hs_ref, gw_ref, logits_ref, combine_ref):
    hs = hs_ref[...]
    gw = gw_ref[...]
    logits = jax.lax.dot_general(
        hs, gw, (((1,), (1,)), ((), ())), preferred_element_type=jnp.float32)
    logits_ref[...] = logits

    ids = jax.lax.broadcasted_iota(jnp.int32, (T, E), 1)
    m0 = jnp.max(logits, axis=1, keepdims=True)
    idx0 = jnp.min(jnp.where(logits == m0, ids, E), axis=1, keepdims=True)
    l2 = jnp.where(ids == idx0, -jnp.inf, logits)
    m1 = jnp.max(l2, axis=1, keepdims=True)
    idx1 = jnp.min(jnp.where(l2 == m1, ids, E), axis=1, keepdims=True)
    w1w = 1.0 / (1.0 + jnp.exp(m0 - m1))
    w0w = 1.0 - w1w
    combine_ref[...] = (w0w * (ids == idx0).astype(jnp.float32)
                        + w1w * (ids == idx1).astype(jnp.float32))


def _expert_body(hs_ref, w1_ref, w2_ref, comb_ref, out_ref):
    e = pl.program_id(0)
    hs = hs_ref[...]
    a = jax.lax.dot_general(
        hs, w1_ref[0], (((1,), (1,)), ((), ())),
        preferred_element_type=jnp.float32)
    g = a[:, :F]
    u = a[:, F:]
    act = (g * jax.lax.logistic(g) * u).astype(jnp.bfloat16)
    y = jax.lax.dot_general(
        act, w2_ref[0], (((1,), (1,)), ((), ())),
        preferred_element_type=jnp.float32)
    ids = jax.lax.broadcasted_iota(jnp.int32, (T, E), 1)
    col = jnp.sum(jnp.where(ids == e, comb_ref[...], 0.0), axis=1,
                  keepdims=True)
    contrib = col * y

    @pl.when(e == 0)
    def _():
        out_ref[...] = contrib

    @pl.when(e > 0)
    def _():
        out_ref[...] += contrib


@jax.jit
def kernel(hidden_states, gate_w, w1, w2):
    logits, combine = pl.pallas_call(
        _router_body,
        out_shape=[
            jax.ShapeDtypeStruct((T, E), jnp.float32),
            jax.ShapeDtypeStruct((T, E), jnp.float32),
        ],
    )(hidden_states, gate_w)

    hs_b = hidden_states.astype(jnp.bfloat16)
    w1_b = w1.astype(jnp.bfloat16)
    w2_b = w2.astype(jnp.bfloat16)

    out = pl.pallas_call(
        _expert_body,
        grid=(E,),
        in_specs=[
            pl.BlockSpec((T, H), lambda e: (0, 0)),
            pl.BlockSpec((1, 2 * F, H), lambda e: (e, 0, 0)),
            pl.BlockSpec((1, H, F), lambda e: (e, 0, 0)),
            pl.BlockSpec((T, E), lambda e: (0, 0)),
        ],
        out_specs=pl.BlockSpec((T, H), lambda e: (0, 0)),
        out_shape=jax.ShapeDtypeStruct((T, H), jnp.float32),
    )(hs_b, w1_b, w2_b, combine)

    return (out, logits)


# speedup vs baseline: 1.5073x; 1.1060x over previous
import functools

import jax
import jax.numpy as jnp
from jax import lax
from jax.experimental import pallas as pl
from jax.experimental.pallas import tpu as pltpu
from jax.experimental.pallas import tpu_sc as plsc

T, H, F, E, TOP_K = 2048, 1024, 1024, 8, 2

BLK = 256
NMAX = T * TOP_K + E * BLK
NB = NMAX // BLK

NC, NS, L = 2, 16, 16
NW = NC * NS
TPW = T // NW
CH = 32
NCH = TPW // CH


def _router_body(hs_ref, gw_ref, logits_ref, pos0_ref, pos1_ref,
                 w0_ref, w1_ref, be_ref):
    hs = hs_ref[...]
    gw = gw_ref[...]
    logits = lax.dot_general(hs, gw, (((1,), (1,)), ((), ())),
                             preferred_element_type=jnp.float32)
    logits_ref[...] = logits

    ids = lax.broadcasted_iota(jnp.int32, (T, E), 1)
    m0 = jnp.max(logits, axis=1, keepdims=True)
    idx0 = jnp.min(jnp.where(logits == m0, ids, E), axis=1, keepdims=True)
    l2 = jnp.where(ids == idx0, -jnp.inf, logits)
    m1 = jnp.max(l2, axis=1, keepdims=True)
    idx1 = jnp.min(jnp.where(l2 == m1, ids, E), axis=1, keepdims=True)
    w1w = 1.0 / (1.0 + jnp.exp(m0 - m1))
    w0w = 1.0 - w1w
    w0_ref[...] = jnp.broadcast_to(w0w, (T, L))
    w1_ref[...] = jnp.broadcast_to(w1w, (T, L))

    oh0 = ids == idx0
    oh1 = ids == idx1
    sel = oh0.astype(jnp.int32) + oh1.astype(jnp.int32)

    csum = sel
    k = 1
    while k < T:
        csum = csum + jnp.pad(csum[:-k, :], ((k, 0), (0, 0)))
        k *= 2
    counts = csum[T - 1:T, :]
    pc = ((counts + (BLK - 1)) // BLK) * BLK
    ends = pc
    k = 1
    while k < E:
        ends = ends + jnp.pad(ends[:, :-k], ((0, 0), (k, 0)))
        k *= 2
    offs = ends - pc

    posfull = offs + csum - 1
    pos0_ref[...] = jnp.sum(jnp.where(oh0, posfull, 0), axis=1, keepdims=True)
    pos1_ref[...] = jnp.sum(jnp.where(oh1, posfull, 0), axis=1, keepdims=True)

    bstart = lax.broadcasted_iota(jnp.int32, (NB, E), 0) * BLK
    be = jnp.sum((jnp.broadcast_to(ends, (NB, E)) <= bstart).astype(jnp.int32),
                 axis=1, keepdims=True)
    be_ref[...] = jnp.minimum(be, E - 1)


def _gemm_body(be_ref, x_ref, w1_ref, w2_ref, y_ref):
    x = x_ref[...].astype(jnp.bfloat16)
    a = lax.dot_general(x, w1_ref[0], (((1,), (1,)), ((), ())),
                        preferred_element_type=jnp.float32)
    g = a[:, :F]
    u = a[:, F:]
    act = (g * lax.logistic(g) * u).astype(jnp.bfloat16)
    y_ref[...] = lax.dot_general(act, w2_ref[0], (((1,), (1,)), ((), ())),
                                 preferred_element_type=jnp.float32)


def _scatter_body(hs_hbm, pos0_hbm, pos1_hbm, xs_hbm, i0_v, i1_v, rows_v, sem):
    wid = lax.axis_index("s") * NC + lax.axis_index("c")
    for j in range(NCH):
        base = wid * TPW + j * CH
        pltpu.sync_copy(pos0_hbm.at[wid, j], i0_v)
        pltpu.sync_copy(pos1_hbm.at[wid, j], i1_v)
        pltpu.sync_copy(hs_hbm.at[pl.ds(base, CH)], rows_v)
        c0 = pltpu.async_copy(rows_v, xs_hbm.at[i0_v], sem)
        c1 = pltpu.async_copy(rows_v, xs_hbm.at[i1_v], sem)
        c0.wait()
        c1.wait()


def _combine_body(y_hbm, pos0_hbm, pos1_hbm, w0_hbm, w1_hbm, out_hbm,
                  p0_v, p1_v, w0_v, w1_v, buf0, buf1, ob, sem0, sem1):
    wid = lax.axis_index("s") * NC + lax.axis_index("c")
    for j in range(NCH):
        base = wid * TPW + j * CH
        pltpu.sync_copy(pos0_hbm.at[wid, j], p0_v)
        pltpu.sync_copy(pos1_hbm.at[wid, j], p1_v)
        pltpu.sync_copy(w0_hbm.at[wid, j], w0_v)
        pltpu.sync_copy(w1_hbm.at[wid, j], w1_v)
        c0 = pltpu.async_copy(y_hbm.at[p0_v], buf0, sem0)
        c1 = pltpu.async_copy(y_hbm.at[p1_v], buf1, sem1)
        c0.wait()
        c1.wait()

        def row_body(i, _):
            s0 = w0_v[i]
            s1 = w1_v[i]
            for c in range(H // L):
                sl = pl.ds(c * L, L)
                ob[i, sl] = s0 * buf0[i, sl] + s1 * buf1[i, sl]
            return 0

        lax.fori_loop(0, CH, row_body, 0)
        pltpu.sync_copy(ob, out_hbm.at[pl.ds(base, CH)])


@jax.jit
def kernel(hidden_states, gate_w, w1, w2):
    logits, pos0, pos1, w0c, w1c, be = pl.pallas_call(
        _router_body,
        out_shape=[
            jax.ShapeDtypeStruct((T, E), jnp.float32),
            jax.ShapeDtypeStruct((T, 1), jnp.int32),
            jax.ShapeDtypeStruct((T, 1), jnp.int32),
            jax.ShapeDtypeStruct((T, L), jnp.float32),
            jax.ShapeDtypeStruct((T, L), jnp.float32),
            jax.ShapeDtypeStruct((NB, 1), jnp.int32),
        ],
    )(hidden_states, gate_w)

    pos0r = pos0.reshape(NW, NCH, CH)
    pos1r = pos1.reshape(NW, NCH, CH)
    w0r = w0c.reshape(NW, NCH, CH, L)
    w1r = w1c.reshape(NW, NCH, CH, L)
    be_flat = be.reshape(NB)

    mesh = plsc.VectorSubcoreMesh(core_axis_name="c", subcore_axis_name="s")

    x_sorted = pl.kernel(
        _scatter_body,
        out_type=jax.ShapeDtypeStruct((NMAX, H), jnp.float32),
        mesh=mesh,
        scratch_types=[
            pltpu.VMEM((CH,), jnp.int32),
            pltpu.VMEM((CH,), jnp.int32),
            pltpu.VMEM((CH, H), jnp.float32),
            pltpu.SemaphoreType.DMA,
        ],
    )(hidden_states, pos0r, pos1r)

    w1_b = w1.astype(jnp.bfloat16)
    w2_b = w2.astype(jnp.bfloat16)

    y = pl.pallas_call(
        _gemm_body,
        grid_spec=pltpu.PrefetchScalarGridSpec(
            num_scalar_prefetch=1,
            grid=(NB,),
            in_specs=[
                pl.BlockSpec((BLK, H), lambda i, be_s: (i, 0)),
                pl.BlockSpec((1, 2 * F, H), lambda i, be_s: (be_s[i], 0, 0)),
                pl.BlockSpec((1, H, F), lambda i, be_s: (be_s[i], 0, 0)),
            ],
            out_specs=pl.BlockSpec((BLK, H), lambda i, be_s: (i, 0)),
        ),
        out_shape=jax.ShapeDtypeStruct((NMAX, H), jnp.float32),
    )(be_flat, x_sorted, w1_b, w2_b)

    out = pl.kernel(
        _combine_body,
        out_type=jax.ShapeDtypeStruct((T, H), jnp.float32),
        mesh=mesh,
        scratch_types=[
            pltpu.VMEM((CH,), jnp.int32),
            pltpu.VMEM((CH,), jnp.int32),
            pltpu.VMEM((CH, L), jnp.float32),
            pltpu.VMEM((CH, L), jnp.float32),
            pltpu.VMEM((CH, H), jnp.float32),
            pltpu.VMEM((CH, H), jnp.float32),
            pltpu.VMEM((CH, H), jnp.float32),
            pltpu.SemaphoreType.DMA,
            pltpu.SemaphoreType.DMA,
        ],
    )(y, pos0r, pos1r, w0r, w1r)

    return (out, logits)


# speedup vs baseline: 1.7961x; 1.1916x over previous
import functools

import jax
import jax.numpy as jnp
from jax import lax
from jax.experimental import pallas as pl
from jax.experimental.pallas import tpu as pltpu
from jax.experimental.pallas import tpu_sc as plsc

T, H, F, E, TOP_K = 2048, 1024, 1024, 8, 2

BLK = 256
NMAX = T * TOP_K + E * BLK
NB = NMAX // BLK

NC, NS, L = 2, 16, 16
NW = NC * NS
TPW = T // NW
CH = 32
NCH = TPW // CH


def _router_body(hs_ref, gw_ref, logits_ref, pos0_ref, pos1_ref,
                 w0_ref, w1_ref, be_ref):
    hs = hs_ref[...]
    gw = gw_ref[...]
    logits = lax.dot_general(hs, gw, (((1,), (1,)), ((), ())),
                             preferred_element_type=jnp.float32)
    logits_ref[...] = logits

    ids = lax.broadcasted_iota(jnp.int32, (T, E), 1)
    m0 = jnp.max(logits, axis=1, keepdims=True)
    idx0 = jnp.min(jnp.where(logits == m0, ids, E), axis=1, keepdims=True)
    l2 = jnp.where(ids == idx0, -jnp.inf, logits)
    m1 = jnp.max(l2, axis=1, keepdims=True)
    idx1 = jnp.min(jnp.where(l2 == m1, ids, E), axis=1, keepdims=True)
    w1w = 1.0 / (1.0 + jnp.exp(m0 - m1))
    w0w = 1.0 - w1w
    w0_ref[...] = jnp.broadcast_to(w0w, (T, L))
    w1_ref[...] = jnp.broadcast_to(w1w, (T, L))

    oh0 = ids == idx0
    oh1 = ids == idx1
    sel = oh0.astype(jnp.int32) + oh1.astype(jnp.int32)

    csum = sel
    k = 1
    while k < T:
        csum = csum + jnp.pad(csum[:-k, :], ((k, 0), (0, 0)))
        k *= 2
    counts = csum[T - 1:T, :]
    pc = ((counts + (BLK - 1)) // BLK) * BLK
    ends = pc
    k = 1
    while k < E:
        ends = ends + jnp.pad(ends[:, :-k], ((0, 0), (k, 0)))
        k *= 2
    offs = ends - pc

    posfull = offs + csum - 1
    pos0_ref[...] = jnp.sum(jnp.where(oh0, posfull, 0), axis=1, keepdims=True)
    pos1_ref[...] = jnp.sum(jnp.where(oh1, posfull, 0), axis=1, keepdims=True)

    bstart = lax.broadcasted_iota(jnp.int32, (NB, E), 0) * BLK
    be = jnp.sum((jnp.broadcast_to(ends, (NB, E)) <= bstart).astype(jnp.int32),
                 axis=1, keepdims=True)
    be_ref[...] = jnp.minimum(be, E - 1)


def _gemm_body(be_ref, x_ref, w1_ref, w2_ref, y_ref):
    x = x_ref[...].astype(jnp.bfloat16)
    w1b = w1_ref[0].astype(jnp.bfloat16)
    a = lax.dot_general(x, w1b, (((1,), (1,)), ((), ())),
                        preferred_element_type=jnp.float32)
    g = a[:, :F]
    u = a[:, F:]
    act = (g * lax.logistic(g) * u).astype(jnp.bfloat16)
    w2b = w2_ref[0].astype(jnp.bfloat16)
    y_ref[...] = lax.dot_general(act, w2b, (((1,), (1,)), ((), ())),
                                 preferred_element_type=jnp.float32)


def _scatter_body(hs_hbm, pos0_hbm, pos1_hbm, xs_hbm, i0_v, i1_v, rows_v, sem):
    wid = lax.axis_index("s") * NC + lax.axis_index("c")
    for j in range(NCH):
        base = wid * TPW + j * CH
        pltpu.sync_copy(pos0_hbm.at[wid, j], i0_v)
        pltpu.sync_copy(pos1_hbm.at[wid, j], i1_v)
        pltpu.sync_copy(hs_hbm.at[pl.ds(base, CH)], rows_v)
        c0 = pltpu.async_copy(rows_v, xs_hbm.at[i0_v], sem)
        c1 = pltpu.async_copy(rows_v, xs_hbm.at[i1_v], sem)
        c0.wait()
        c1.wait()


def _combine_body(y_hbm, pos0_hbm, pos1_hbm, w0_hbm, w1_hbm, out_hbm,
                  p0_v, p1_v, w0_v, w1_v, buf0, buf1, ob, sem0, sem1):
    wid = lax.axis_index("s") * NC + lax.axis_index("c")
    for j in range(NCH):
        base = wid * TPW + j * CH
        pltpu.sync_copy(pos0_hbm.at[wid, j], p0_v)
        pltpu.sync_copy(pos1_hbm.at[wid, j], p1_v)
        pltpu.sync_copy(w0_hbm.at[wid, j], w0_v)
        pltpu.sync_copy(w1_hbm.at[wid, j], w1_v)
        c0 = pltpu.async_copy(y_hbm.at[p0_v], buf0, sem0)
        c1 = pltpu.async_copy(y_hbm.at[p1_v], buf1, sem1)
        c0.wait()
        c1.wait()

        def row_body(i, _):
            s0 = w0_v[i]
            s1 = w1_v[i]
            for c in range(H // L):
                sl = pl.ds(c * L, L)
                ob[i, sl] = s0 * buf0[i, sl] + s1 * buf1[i, sl]
            return 0

        lax.fori_loop(0, CH, row_body, 0)
        pltpu.sync_copy(ob, out_hbm.at[pl.ds(base, CH)])


@jax.jit
def kernel(hidden_states, gate_w, w1, w2):
    logits, pos0, pos1, w0c, w1c, be = pl.pallas_call(
        _router_body,
        out_shape=[
            jax.ShapeDtypeStruct((T, E), jnp.float32),
            jax.ShapeDtypeStruct((T, 1), jnp.int32),
            jax.ShapeDtypeStruct((T, 1), jnp.int32),
            jax.ShapeDtypeStruct((T, L), jnp.float32),
            jax.ShapeDtypeStruct((T, L), jnp.float32),
            jax.ShapeDtypeStruct((NB, 1), jnp.int32),
        ],
    )(hidden_states, gate_w)

    pos0r = pos0.reshape(NW, NCH, CH)
    pos1r = pos1.reshape(NW, NCH, CH)
    w0r = w0c.reshape(NW, NCH, CH, L)
    w1r = w1c.reshape(NW, NCH, CH, L)
    be_flat = be.reshape(NB)

    mesh = plsc.VectorSubcoreMesh(core_axis_name="c", subcore_axis_name="s")

    x_sorted = pl.kernel(
        _scatter_body,
        out_type=jax.ShapeDtypeStruct((NMAX, H), jnp.float32),
        mesh=mesh,
        scratch_types=[
            pltpu.VMEM((CH,), jnp.int32),
            pltpu.VMEM((CH,), jnp.int32),
            pltpu.VMEM((CH, H), jnp.float32),
            pltpu.SemaphoreType.DMA,
        ],
    )(hidden_states, pos0r, pos1r)

    y = pl.pallas_call(
        _gemm_body,
        grid_spec=pltpu.PrefetchScalarGridSpec(
            num_scalar_prefetch=1,
            grid=(NB,),
            in_specs=[
                pl.BlockSpec((BLK, H), lambda i, be_s: (i, 0)),
                pl.BlockSpec((1, 2 * F, H), lambda i, be_s: (be_s[i], 0, 0)),
                pl.BlockSpec((1, H, F), lambda i, be_s: (be_s[i], 0, 0)),
            ],
            out_specs=pl.BlockSpec((BLK, H), lambda i, be_s: (i, 0)),
        ),
        out_shape=jax.ShapeDtypeStruct((NMAX, H), jnp.float32),
    )(be_flat, x_sorted, w1, w2)

    out = pl.kernel(
        _combine_body,
        out_type=jax.ShapeDtypeStruct((T, H), jnp.float32),
        mesh=mesh,
        scratch_types=[
            pltpu.VMEM((CH,), jnp.int32),
            pltpu.VMEM((CH,), jnp.int32),
            pltpu.VMEM((CH, L), jnp.float32),
            pltpu.VMEM((CH, L), jnp.float32),
            pltpu.VMEM((CH, H), jnp.float32),
            pltpu.VMEM((CH, H), jnp.float32),
            pltpu.VMEM((CH, H), jnp.float32),
            pltpu.SemaphoreType.DMA,
            pltpu.SemaphoreType.DMA,
        ],
    )(y, pos0r, pos1r, w0r, w1r)

    return (out, logits)
